# baseline (speedup 1.0000x reference)
import functools

import jax
import jax.numpy as jnp
from jax import lax
from jax.experimental import pallas as pl
from jax.experimental.pallas import tpu as pltpu
from jax.experimental.pallas import tpu_sc as plsc

F32 = jnp.float32

_N = 10000
_E = 320000
_H = 128
_HH = _H // 2
_G = 64

_NC = 2
_NS = 16
_NW = _NC * _NS
_CH = 128
_EP = 327680
_EPT = _EP // _NW
_NCHUNK = _EPT // _CH
_NP = _N + 16
_RPT = 640
_RCH = 160


def _dot_t(a, w):
    return lax.dot_general(a, w, (((1,), (1,)), ((), ())),
                           preferred_element_type=F32)


_NB = 5


def _fill(ref, rows, cols, val):
    groups = cols // 16

    def body(i, c):
        ref[i // groups, pl.ds((i % groups) * 16, 16)] = jnp.full((16,), val, F32)
        return c

    lax.fori_loop(0, rows * groups, body, 0)


def _sc_agg_body(with_counts, *refs):
    if with_counts:
        (u0_hbm, u1_hbm, e_hbm,
         s_out, cnt_out,
         src_blk, dst_blk, zbuf, ones_v, cbuf, acc_sp, cnt_sp,
         *rest) = refs
    else:
        (u0_hbm, u1_hbm, e_hbm,
         s_out,
         src_blk, dst_blk, zbuf, acc_sp, *rest) = refs
    rows_v = rest[:_NB]
    gsem = rest[_NB:2 * _NB]
    ssem = rest[2 * _NB:3 * _NB]
    csem = rest[3 * _NB:]

    cid = lax.axis_index("c")
    sid = lax.axis_index("s")
    w = cid * _NS + sid
    nck = jnp.clip((_E // _CH) - w * _NCHUNK, _NB, _NCHUNK)
    my_rows = pl.ds(jnp.minimum(sid * _RPT, _N - _RPT), _RPT)

    start_row = jnp.minimum(sid * _RPT, _N - _RPT)

    pltpu.sync_copy(e_hbm.at[0, pl.ds(w * _NCHUNK, _NCHUNK)], src_blk)
    pltpu.sync_copy(e_hbm.at[1, pl.ds(w * _NCHUNK, _NCHUNK)], dst_blk)
    if with_counts:
        _fill(ones_v, _CH, 16, 1.0)

    for h, u_hbm in ((0, u0_hbm), (1, u1_hbm)):
        count_pass = with_counts and h == 0
        _fill(zbuf, _RCH, _HH, 0.0)
        for k in range(_RPT // _RCH):
            pltpu.sync_copy(zbuf, acc_sp.at[pl.ds(start_row + k * _RCH, _RCH)])
        if count_pass:
            _fill(cbuf, _RCH, 16, 0.0)
            for k in range(_RPT // _RCH):
                pltpu.sync_copy(cbuf, cnt_sp.at[pl.ds(start_row + k * _RCH,
                                                      _RCH)])
        plsc.subcore_barrier()

        def wait_gather(b):
            pltpu.make_async_copy(u_hbm.at[pl.ds(0, _CH)], rows_v[b],
                                  gsem[b]).wait()

        def start_gather(b, i):
            pltpu.async_copy(u_hbm.at[src_blk.at[i]], rows_v[b], gsem[b])

        def start_scatter(b, i):
            pltpu.async_copy(rows_v[b], acc_sp.at[dst_blk.at[i]], ssem[b],
                             add=True)
            if count_pass:
                pltpu.async_copy(ones_v, cnt_sp.at[dst_blk.at[i]], csem[b],
                                 add=True)

        def wait_scatter(b):
            pltpu.make_async_copy(rows_v[b], acc_sp.at[pl.ds(0, _CH)],
                                  ssem[b]).wait()
            if count_pass:
                pltpu.make_async_copy(ones_v, cnt_sp.at[pl.ds(0, _CH)],
                                      csem[b]).wait()

        for b in range(_NB):
            start_gather(b, b)

        def window(win, carry):
            for b in range(_NB):
                i = win * _NB + b
                wait_gather(b)
                start_scatter(b, i)
                j = i + 3
                bj = (b + 3) % _NB

                @pl.when(jnp.logical_and(j >= _NB, j < nck))
                def _():
                    wait_scatter(bj)
                    start_gather(bj, j)

            return carry

        lax.fori_loop(0, nck // _NB, window, 0)
        for b in range(_NB):
            wait_scatter(b)
        plsc.subcore_barrier()

        for k in range(_RPT // _RCH):
            rows_k = pl.ds(start_row + k * _RCH, _RCH)
            pltpu.sync_copy(acc_sp.at[rows_k], zbuf)
            pltpu.sync_copy(zbuf, s_out.at[cid, rows_k, pl.ds(h * _HH, _HH)])
        if count_pass:
            for k in range(_RPT // _RCH):
                rows_k = pl.ds(start_row + k * _RCH, _RCH)
                pltpu.sync_copy(cnt_sp.at[rows_k], cbuf)
                pltpu.sync_copy(cbuf, cnt_out.at[cid, rows_k])


def _make_sc_agg(with_counts):
    mesh = plsc.VectorSubcoreMesh(core_axis_name="c", subcore_axis_name="s",
                                  num_cores=_NC, num_subcores=_NS)
    out_type = [jax.ShapeDtypeStruct((_NC, _N, _H), F32)]
    scratch = [
        pltpu.VMEM((_NCHUNK, _CH), jnp.int32),
        pltpu.VMEM((_NCHUNK, _CH), jnp.int32),
        pltpu.VMEM((_RCH, _HH), F32),
    ]
    if with_counts:
        out_type.append(jax.ShapeDtypeStruct((_NC, _N, 16), F32))
        scratch += [
            pltpu.VMEM((_CH, 16), F32),
            pltpu.VMEM((_RCH, 16), F32),
        ]
    scratch.append(pltpu.VMEM_SHARED((_NP, _HH), F32))
    if with_counts:
        scratch.append(pltpu.VMEM_SHARED((_NP, 16), F32))
    scratch += [pltpu.VMEM((_CH, _HH), F32) for _ in range(_NB)]
    scratch += [pltpu.SemaphoreType.DMA for _ in range(3 * _NB)]

    return pl.kernel(
        functools.partial(_sc_agg_body, with_counts),
        out_type=tuple(out_type),
        mesh=mesh,
        scratch_types=scratch,
        compiler_params=pltpu.CompilerParams(use_tc_tiling_on_sc=False),
    )


def _proj_body(x_ref, wrel_ref, wroot_ref, b_ref, u0_ref, u1_ref, r_ref):
    xb = x_ref[...]
    u = _dot_t(xb, wrel_ref[...])
    u0_ref[...] = u[:, :_HH]
    u1_ref[...] = u[:, _HH:]
    r_ref[...] = _dot_t(xb, wroot_ref[...]) + b_ref[...]


def _combine(s_ref, c16_ref, r_ref):
    ssum = s_ref[0] + s_ref[1]
    csum = jnp.sum(c16_ref[...], axis=(0, 2))
    inv = 16.0 / jnp.maximum(csum, 16.0)
    return jnp.maximum(ssum * inv[:, None] + r_ref[...], 0.0)


def _mid_body(s_ref, c16_ref, r_ref, wrel_ref, wroot_ref, b_ref,
              u0_ref, u1_ref, r2_ref):
    h = _combine(s_ref, c16_ref, r_ref)
    u = _dot_t(h, wrel_ref[...])
    u0_ref[...] = u[:, :_HH]
    u1_ref[...] = u[:, _HH:]
    r2_ref[...] = _dot_t(h, wroot_ref[...]) + b_ref[...]


def _pool_body(s_ref, c16_ref, r_ref, batch_ref, wcls_ref, bcls_ref, out_ref):
    h = _combine(s_ref, c16_ref, r_ref)
    gid = lax.broadcasted_iota(jnp.int32, (_G, _N), 0)
    mask = (batch_ref[...] == gid).astype(F32)
    seg = lax.dot_general(mask, h, (((1,), (0,)), ((), ())),
                          preferred_element_type=F32)
    cnt = jnp.sum(mask, axis=1)
    pooled = seg / jnp.maximum(cnt, 1.0)[:, None]
    out_ref[...] = _dot_t(pooled, wcls_ref[...]) + bcls_ref[...]


_ROWS_BLK = 1000


def _proj_call(x, wrel, wroot, b):
    n = x.shape[0]
    grid = n // _ROWS_BLK
    full = pl.BlockSpec((_H, _H), lambda i: (0, 0))
    rows = pl.BlockSpec((_ROWS_BLK, _H), lambda i: (i, 0))
    rows_h = pl.BlockSpec((_ROWS_BLK, _HH), lambda i: (i, 0))
    return pl.pallas_call(
        _proj_body,
        grid=(grid,),
        in_specs=[rows, full, full, pl.BlockSpec((1, _H), lambda i: (0, 0))],
        out_specs=[rows_h, rows_h, rows],
        out_shape=[jax.ShapeDtypeStruct((n, _HH), F32)] * 2
        + [jax.ShapeDtypeStruct((n, _H), F32)],
    )(x, wrel, wroot, b)


def _mid_call(s_part, c16, r, wrel, wroot, b):
    grid = _N // _ROWS_BLK
    full = pl.BlockSpec((_H, _H), lambda i: (0, 0))
    rows = pl.BlockSpec((_ROWS_BLK, _H), lambda i: (i, 0))
    rows_h = pl.BlockSpec((_ROWS_BLK, _HH), lambda i: (i, 0))
    return pl.pallas_call(
        _mid_body,
        grid=(grid,),
        in_specs=[
            pl.BlockSpec((_NC, _ROWS_BLK, _H), lambda i: (0, i, 0)),
            pl.BlockSpec((_NC, _ROWS_BLK, 16), lambda i: (0, i, 0)),
            rows, full, full, pl.BlockSpec((1, _H), lambda i: (0, 0)),
        ],
        out_specs=[rows_h, rows_h, rows],
        out_shape=[jax.ShapeDtypeStruct((_N, _HH), F32)] * 2
        + [jax.ShapeDtypeStruct((_N, _H), F32)],
    )(s_part, c16, r, wrel, wroot, b)


def _pool_call(s_part, c16, r, batch2d, wcls, bcls):
    return pl.pallas_call(
        _pool_body,
        out_shape=jax.ShapeDtypeStruct((_G, _H), F32),
    )(s_part, c16, r, batch2d, wcls, bcls)


def kernel(x, edge_index, batch, W_rel1, b_rel1, W_root1,
           W_rel2, b_rel2, W_root2, W_cls, b_cls):
    npad = _EP - _E
    pad = jnp.concatenate(
        [jnp.zeros((1, npad), jnp.int32),
         jnp.full((1, npad), _N, jnp.int32)], axis=0)
    epad = jnp.concatenate([edge_index, pad], axis=1)
    epad = epad.reshape(2, _EP // _CH, _CH)
    batch2d = batch.reshape(1, _N)

    sc_agg1 = _make_sc_agg(True)
    sc_agg2 = _make_sc_agg(False)

    u0, u1, r1 = _proj_call(x, W_rel1, W_root1, b_rel1.reshape(1, _H))
    s1, c16 = sc_agg1(u0, u1, epad)
    v0, v1, r2 = _mid_call(s1, c16, r1, W_rel2, W_root2, b_rel2.reshape(1, _H))
    s2 = sc_agg2(v0, v1, epad)
    if isinstance(s2, (tuple, list)):
        s2 = s2[0]
    return _pool_call(s2, c16, r2, batch2d, W_cls, b_cls.reshape(1, _H))


# speedup vs baseline: 1.0305x; 1.0305x over previous
import functools

import jax
import jax.numpy as jnp
from jax import lax
from jax.experimental import pallas as pl
from jax.experimental.pallas import tpu as pltpu
from jax.experimental.pallas import tpu_sc as plsc

F32 = jnp.float32

_N = 10000
_E = 320000
_H = 128
_HH = _H // 2
_G = 64

_NC = 2
_NS = 16
_NW = _NC * _NS
_CH = 128
_EP = 327680
_EPT = _EP // _NW
_NCHUNK = _EPT // _CH
_NP = _N + 16
_RPT = 640
_RCH = 160


def _dot_t(a, w):
    return lax.dot_general(a.astype(jnp.bfloat16), w.astype(jnp.bfloat16),
                           (((1,), (1,)), ((), ())),
                           preferred_element_type=F32)


_NB = 4


def _fill(ref, rows, cols, val):
    groups = cols // 16

    def body(i, c):
        ref[i // groups, pl.ds((i % groups) * 16, 16)] = jnp.full((16,), val, F32)
        return c

    lax.fori_loop(0, rows * groups, body, 0)


def _sc_agg_body(with_counts, *refs):
    if with_counts:
        (u0_hbm, u1_hbm, e_hbm,
         s_out, cnt_out,
         src_blk, dst_blk, zbuf, ones_v, cbuf, acc_sp, cnt_sp,
         *rest) = refs
    else:
        (u0_hbm, u1_hbm, e_hbm,
         s_out,
         src_blk, dst_blk, zbuf, acc_sp, *rest) = refs
    rows_v = rest[:_NB]
    gsem = rest[_NB:2 * _NB]
    ssem = rest[2 * _NB:3 * _NB]
    csem = rest[3 * _NB:]

    cid = lax.axis_index("c")
    sid = lax.axis_index("s")
    w = cid * _NS + sid
    nck = jnp.clip((_E // _CH) - w * _NCHUNK, _NB, _NCHUNK)
    my_rows = pl.ds(jnp.minimum(sid * _RPT, _N - _RPT), _RPT)

    start_row = jnp.minimum(sid * _RPT, _N - _RPT)

    pltpu.sync_copy(e_hbm.at[0, pl.ds(w * _NCHUNK, _NCHUNK)], src_blk)
    pltpu.sync_copy(e_hbm.at[1, pl.ds(w * _NCHUNK, _NCHUNK)], dst_blk)
    if with_counts:
        _fill(ones_v, _CH, 16, 1.0)

    for h, u_hbm in ((0, u0_hbm), (1, u1_hbm)):
        count_pass = with_counts and h == 0
        _fill(zbuf, _RCH, _HH, 0.0)
        for k in range(_RPT // _RCH):
            pltpu.sync_copy(zbuf, acc_sp.at[pl.ds(start_row + k * _RCH, _RCH)])
        if count_pass:
            _fill(cbuf, _RCH, 16, 0.0)
            for k in range(_RPT // _RCH):
                pltpu.sync_copy(cbuf, cnt_sp.at[pl.ds(start_row + k * _RCH,
                                                      _RCH)])
        plsc.subcore_barrier()

        def wait_gather(b):
            pltpu.make_async_copy(u_hbm.at[pl.ds(0, _CH)], rows_v[b],
                                  gsem[b]).wait()

        def start_gather(b, i):
            pltpu.async_copy(u_hbm.at[src_blk.at[i]], rows_v[b], gsem[b])

        def scatter(b, i):
            pltpu.sync_copy(rows_v[b], acc_sp.at[dst_blk.at[i]], add=True)
            if count_pass:
                pltpu.sync_copy(ones_v, cnt_sp.at[dst_blk.at[i]], add=True)

        for b in range(_NB):
            start_gather(b, b)

        def window(win, carry):
            for b in range(_NB):
                i = win * _NB + b
                wait_gather(b)
                scatter(b, i)
                start_gather(b, i + _NB)
            return carry

        lax.fori_loop(0, nck // _NB - 1, window, 0)
        for b in range(_NB):
            i = nck - _NB + b
            wait_gather(b)
            scatter(b, i)
        plsc.subcore_barrier()

        for k in range(_RPT // _RCH):
            rows_k = pl.ds(start_row + k * _RCH, _RCH)
            pltpu.sync_copy(acc_sp.at[rows_k], zbuf)
            pltpu.sync_copy(zbuf, s_out.at[cid, rows_k, pl.ds(h * _HH, _HH)])
        if count_pass:
            for k in range(_RPT // _RCH):
                rows_k = pl.ds(start_row + k * _RCH, _RCH)
                pltpu.sync_copy(cnt_sp.at[rows_k], cbuf)
                pltpu.sync_copy(cbuf, cnt_out.at[cid, rows_k])


def _make_sc_agg(with_counts):
    mesh = plsc.VectorSubcoreMesh(core_axis_name="c", subcore_axis_name="s",
                                  num_cores=_NC, num_subcores=_NS)
    out_type = [jax.ShapeDtypeStruct((_NC, _N, _H), F32)]
    scratch = [
        pltpu.VMEM((_NCHUNK, _CH), jnp.int32),
        pltpu.VMEM((_NCHUNK, _CH), jnp.int32),
        pltpu.VMEM((_RCH, _HH), F32),
    ]
    if with_counts:
        out_type.append(jax.ShapeDtypeStruct((_NC, _N, 16), F32))
        scratch += [
            pltpu.VMEM((_CH, 16), F32),
            pltpu.VMEM((_RCH, 16), F32),
        ]
    scratch.append(pltpu.VMEM_SHARED((_NP, _HH), F32))
    if with_counts:
        scratch.append(pltpu.VMEM_SHARED((_NP, 16), F32))
    scratch += [pltpu.VMEM((_CH, _HH), F32) for _ in range(_NB)]
    scratch += [pltpu.SemaphoreType.DMA for _ in range(3 * _NB)]

    return pl.kernel(
        functools.partial(_sc_agg_body, with_counts),
        out_type=tuple(out_type),
        mesh=mesh,
        scratch_types=scratch,
        compiler_params=pltpu.CompilerParams(use_tc_tiling_on_sc=False),
    )


def _proj_body(x_ref, wrel_ref, wroot_ref, b_ref, u0_ref, u1_ref, r_ref):
    xb = x_ref[...]
    u = _dot_t(xb, wrel_ref[...])
    u0_ref[...] = u[:, :_HH]
    u1_ref[...] = u[:, _HH:]
    r_ref[...] = _dot_t(xb, wroot_ref[...]) + b_ref[...]


def _combine(s_ref, c16_ref, r_ref):
    ssum = s_ref[0] + s_ref[1]
    csum = jnp.sum(c16_ref[...], axis=(0, 2))
    inv = 16.0 / jnp.maximum(csum, 16.0)
    return jnp.maximum(ssum * inv[:, None] + r_ref[...], 0.0)


def _mid_body(s_ref, c16_ref, r_ref, wrel_ref, wroot_ref, b_ref,
              u0_ref, u1_ref, r2_ref):
    h = _combine(s_ref, c16_ref, r_ref)
    u = _dot_t(h, wrel_ref[...])
    u0_ref[...] = u[:, :_HH]
    u1_ref[...] = u[:, _HH:]
    r2_ref[...] = _dot_t(h, wroot_ref[...]) + b_ref[...]


def _pool_body(s_ref, c16_ref, r_ref, batch_ref, wcls_ref, bcls_ref, out_ref):
    h = _combine(s_ref, c16_ref, r_ref)
    gid = lax.broadcasted_iota(jnp.int32, (_G, _N), 0)
    mask = (batch_ref[...] == gid).astype(F32)
    seg = lax.dot_general(mask.astype(jnp.bfloat16), h.astype(jnp.bfloat16),
                          (((1,), (0,)), ((), ())),
                          preferred_element_type=F32)
    cnt = jnp.sum(mask, axis=1)
    pooled = seg / jnp.maximum(cnt, 1.0)[:, None]
    out_ref[...] = _dot_t(pooled, wcls_ref[...]) + bcls_ref[...]


_ROWS_BLK = 1000


def _proj_call(x, wrel, wroot, b):
    n = x.shape[0]
    grid = n // _ROWS_BLK
    full = pl.BlockSpec((_H, _H), lambda i: (0, 0))
    rows = pl.BlockSpec((_ROWS_BLK, _H), lambda i: (i, 0))
    rows_h = pl.BlockSpec((_ROWS_BLK, _HH), lambda i: (i, 0))
    return pl.pallas_call(
        _proj_body,
        grid=(grid,),
        in_specs=[rows, full, full, pl.BlockSpec((1, _H), lambda i: (0, 0))],
        out_specs=[rows_h, rows_h, rows],
        out_shape=[jax.ShapeDtypeStruct((n, _HH), F32)] * 2
        + [jax.ShapeDtypeStruct((n, _H), F32)],
    )(x, wrel, wroot, b)


def _mid_call(s_part, c16, r, wrel, wroot, b):
    grid = _N // _ROWS_BLK
    full = pl.BlockSpec((_H, _H), lambda i: (0, 0))
    rows = pl.BlockSpec((_ROWS_BLK, _H), lambda i: (i, 0))
    rows_h = pl.BlockSpec((_ROWS_BLK, _HH), lambda i: (i, 0))
    return pl.pallas_call(
        _mid_body,
        grid=(grid,),
        in_specs=[
            pl.BlockSpec((_NC, _ROWS_BLK, _H), lambda i: (0, i, 0)),
            pl.BlockSpec((_NC, _ROWS_BLK, 16), lambda i: (0, i, 0)),
            rows, full, full, pl.BlockSpec((1, _H), lambda i: (0, 0)),
        ],
        out_specs=[rows_h, rows_h, rows],
        out_shape=[jax.ShapeDtypeStruct((_N, _HH), F32)] * 2
        + [jax.ShapeDtypeStruct((_N, _H), F32)],
    )(s_part, c16, r, wrel, wroot, b)


def _pool_call(s_part, c16, r, batch2d, wcls, bcls):
    return pl.pallas_call(
        _pool_body,
        out_shape=jax.ShapeDtypeStruct((_G, _H), F32),
    )(s_part, c16, r, batch2d, wcls, bcls)


def kernel(x, edge_index, batch, W_rel1, b_rel1, W_root1,
           W_rel2, b_rel2, W_root2, W_cls, b_cls):
    npad = _EP - _E
    pad = jnp.concatenate(
        [jnp.zeros((1, npad), jnp.int32),
         jnp.full((1, npad), _N, jnp.int32)], axis=0)
    epad = jnp.concatenate([edge_index, pad], axis=1)
    epad = epad.reshape(2, _EP // _CH, _CH)
    batch2d = batch.reshape(1, _N)

    sc_agg1 = _make_sc_agg(True)
    sc_agg2 = _make_sc_agg(False)

    u0, u1, r1 = _proj_call(x, W_rel1, W_root1, b_rel1.reshape(1, _H))
    s1, c16 = sc_agg1(u0, u1, epad)
    v0, v1, r2 = _mid_call(s1, c16, r1, W_rel2, W_root2, b_rel2.reshape(1, _H))
    s2 = sc_agg2(v0, v1, epad)
    if isinstance(s2, (tuple, list)):
        s2 = s2[0]
    return _pool_call(s2, c16, r2, batch2d, W_cls, b_cls.reshape(1, _H))


# speedup vs baseline: 1.0460x; 1.0150x over previous
import functools

import jax
import jax.numpy as jnp
from jax import lax
from jax.experimental import pallas as pl
from jax.experimental.pallas import tpu as pltpu
from jax.experimental.pallas import tpu_sc as plsc

F32 = jnp.float32

_N = 10000
_E = 320000
_H = 128
_HH = _H // 2
_G = 64

_NC = 2
_NS = 16
_NW = _NC * _NS
_CH = 128
_EP = 327680
_EPT = _EP // _NW
_NCHUNK = _EPT // _CH
_NP = _N + 16
_RPT = 640
_RCH = 160


def _dot_t(a, w):
    return lax.dot_general(a.astype(jnp.bfloat16), w.astype(jnp.bfloat16),
                           (((1,), (1,)), ((), ())),
                           preferred_element_type=F32)


_NB = 4


def _fill(ref, rows, cols, val):
    groups = cols // 16

    def body(i, c):
        ref[i // groups, pl.ds((i % groups) * 16, 16)] = jnp.full((16,), val, F32)
        return c

    lax.fori_loop(0, rows * groups, body, 0)


def _sc_agg_body(with_counts, *refs):
    if with_counts:
        (u0_hbm, u1_hbm, e_hbm,
         s_out, cnt_out,
         src_blk, dst_blk, zbuf, ones_v, cbuf, acc_sp, cnt_sp,
         *rest) = refs
    else:
        (u0_hbm, u1_hbm, e_hbm,
         s_out,
         src_blk, dst_blk, zbuf, acc_sp, *rest) = refs
    rows_v = rest[:_NB]
    gsem = rest[_NB:2 * _NB]
    ssem = rest[2 * _NB:3 * _NB]
    csem = rest[3 * _NB:]

    cid = lax.axis_index("c")
    sid = lax.axis_index("s")
    w = cid * _NS + sid
    nck = jnp.clip((_E // _CH) - w * _NCHUNK, _NB, _NCHUNK)
    my_rows = pl.ds(jnp.minimum(sid * _RPT, _N - _RPT), _RPT)

    start_row = jnp.minimum(sid * _RPT, _N - _RPT)

    pltpu.sync_copy(e_hbm.at[0, pl.ds(w * _NCHUNK, _NCHUNK)], src_blk)
    pltpu.sync_copy(e_hbm.at[1, pl.ds(w * _NCHUNK, _NCHUNK)], dst_blk)
    if with_counts:
        _fill(ones_v, _CH, 16, 1.0)

    for h, u_hbm in ((0, u0_hbm), (1, u1_hbm)):
        count_pass = with_counts and h == 0
        _fill(zbuf, _RCH, _HH, 0.0)
        for k in range(_RPT // _RCH):
            pltpu.sync_copy(zbuf, acc_sp.at[pl.ds(start_row + k * _RCH, _RCH)])
        if count_pass:
            _fill(cbuf, _RCH, 16, 0.0)
            for k in range(_RPT // _RCH):
                pltpu.sync_copy(cbuf, cnt_sp.at[pl.ds(start_row + k * _RCH,
                                                      _RCH)])
        plsc.subcore_barrier()

        def wait_gather(b):
            pltpu.make_async_copy(u_hbm.at[pl.ds(0, _CH)], rows_v[b],
                                  gsem[b]).wait()

        def start_gather(b, i):
            pltpu.async_copy(u_hbm.at[src_blk.at[i]], rows_v[b], gsem[b])

        def scatter(b, i):
            pltpu.sync_copy(rows_v[b], acc_sp.at[dst_blk.at[i]], add=True)
            if count_pass:
                pltpu.sync_copy(ones_v, cnt_sp.at[dst_blk.at[i]], add=True)

        for b in range(_NB):
            start_gather(b, b)

        def window(win, carry):
            for b in range(_NB):
                i = win * _NB + b
                wait_gather(b)
                scatter(b, i)
                start_gather(b, i + _NB)
            return carry

        lax.fori_loop(0, nck // _NB - 1, window, 0)
        for b in range(_NB):
            i = nck - _NB + b
            wait_gather(b)
            scatter(b, i)
        plsc.subcore_barrier()

        for k in range(_RPT // _RCH):
            rows_k = pl.ds(start_row + k * _RCH, _RCH)
            pltpu.sync_copy(acc_sp.at[rows_k], zbuf)
            pltpu.sync_copy(zbuf, s_out.at[cid, rows_k, pl.ds(h * _HH, _HH)])
        if count_pass:
            for k in range(_RPT // _RCH):
                rows_k = pl.ds(start_row + k * _RCH, _RCH)
                pltpu.sync_copy(cnt_sp.at[rows_k], cbuf)
                pltpu.sync_copy(cbuf, cnt_out.at[cid, rows_k])


def _make_sc_agg(with_counts):
    mesh = plsc.VectorSubcoreMesh(core_axis_name="c", subcore_axis_name="s",
                                  num_cores=_NC, num_subcores=_NS)
    out_type = [jax.ShapeDtypeStruct((_NC, _N, _H), F32)]
    scratch = [
        pltpu.VMEM((_NCHUNK, _CH), jnp.int32),
        pltpu.VMEM((_NCHUNK, _CH), jnp.int32),
        pltpu.VMEM((_RCH, _HH), F32),
    ]
    if with_counts:
        out_type.append(jax.ShapeDtypeStruct((_NC, _N, 16), F32))
        scratch += [
            pltpu.VMEM((_CH, 16), F32),
            pltpu.VMEM((_RCH, 16), F32),
        ]
    scratch.append(pltpu.VMEM_SHARED((_NP, _HH), F32))
    if with_counts:
        scratch.append(pltpu.VMEM_SHARED((_NP, 16), F32))
    scratch += [pltpu.VMEM((_CH, _HH), F32) for _ in range(_NB)]
    scratch += [pltpu.SemaphoreType.DMA for _ in range(3 * _NB)]

    return pl.kernel(
        functools.partial(_sc_agg_body, with_counts),
        out_type=tuple(out_type),
        mesh=mesh,
        scratch_types=scratch,
        compiler_params=pltpu.CompilerParams(use_tc_tiling_on_sc=False),
    )


def _proj_body(x_ref, wrel_ref, wroot_ref, b_ref, u0_ref, u1_ref, r_ref):
    xb = x_ref[...]
    u = _dot_t(xb, wrel_ref[...])
    u0_ref[...] = u[:, :_HH]
    u1_ref[...] = u[:, _HH:]
    r_ref[...] = _dot_t(xb, wroot_ref[...]) + b_ref[...]


def _combine(s_ref, c16_ref, r_ref):
    ssum = s_ref[0] + s_ref[1]
    csum = jnp.sum(c16_ref[...], axis=(0, 2))
    inv = 16.0 / jnp.maximum(csum, 16.0)
    return jnp.maximum(ssum * inv[:, None] + r_ref[...], 0.0)


def _mid_body(s_ref, c16_ref, r_ref, wrel_ref, wroot_ref, b_ref,
              u0_ref, u1_ref, r2_ref):
    h = _combine(s_ref, c16_ref, r_ref)
    u = _dot_t(h, wrel_ref[...])
    u0_ref[...] = u[:, :_HH]
    u1_ref[...] = u[:, _HH:]
    r2_ref[...] = _dot_t(h, wroot_ref[...]) + b_ref[...]


def _pool_body(s_ref, c16_ref, r_ref, batch_ref, wcls_ref, bcls_ref, out_ref):
    h = _combine(s_ref, c16_ref, r_ref)
    gid = lax.broadcasted_iota(jnp.int32, (_G, _N), 0)
    mask = (batch_ref[...] == gid).astype(F32)
    seg = lax.dot_general(mask.astype(jnp.bfloat16), h.astype(jnp.bfloat16),
                          (((1,), (0,)), ((), ())),
                          preferred_element_type=F32)
    cnt = jnp.sum(mask, axis=1)
    pooled = seg / jnp.maximum(cnt, 1.0)[:, None]
    out_ref[...] = _dot_t(pooled, wcls_ref[...]) + bcls_ref[...]


_ROWS_BLK = 2000


def _proj_call(x, wrel, wroot, b):
    n = x.shape[0]
    grid = n // _ROWS_BLK
    full = pl.BlockSpec((_H, _H), lambda i: (0, 0))
    rows = pl.BlockSpec((_ROWS_BLK, _H), lambda i: (i, 0))
    rows_h = pl.BlockSpec((_ROWS_BLK, _HH), lambda i: (i, 0))
    return pl.pallas_call(
        _proj_body,
        grid=(grid,),
        in_specs=[rows, full, full, pl.BlockSpec((1, _H), lambda i: (0, 0))],
        out_specs=[rows_h, rows_h, rows],
        out_shape=[jax.ShapeDtypeStruct((n, _HH), F32)] * 2
        + [jax.ShapeDtypeStruct((n, _H), F32)],
    )(x, wrel, wroot, b)


def _mid_call(s_part, c16, r, wrel, wroot, b):
    grid = _N // _ROWS_BLK
    full = pl.BlockSpec((_H, _H), lambda i: (0, 0))
    rows = pl.BlockSpec((_ROWS_BLK, _H), lambda i: (i, 0))
    rows_h = pl.BlockSpec((_ROWS_BLK, _HH), lambda i: (i, 0))
    return pl.pallas_call(
        _mid_body,
        grid=(grid,),
        in_specs=[
            pl.BlockSpec((_NC, _ROWS_BLK, _H), lambda i: (0, i, 0)),
            pl.BlockSpec((_NC, _ROWS_BLK, 16), lambda i: (0, i, 0)),
            rows, full, full, pl.BlockSpec((1, _H), lambda i: (0, 0)),
        ],
        out_specs=[rows_h, rows_h, rows],
        out_shape=[jax.ShapeDtypeStruct((_N, _HH), F32)] * 2
        + [jax.ShapeDtypeStruct((_N, _H), F32)],
    )(s_part, c16, r, wrel, wroot, b)


def _pool_call(s_part, c16, r, batch2d, wcls, bcls):
    return pl.pallas_call(
        _pool_body,
        out_shape=jax.ShapeDtypeStruct((_G, _H), F32),
    )(s_part, c16, r, batch2d, wcls, bcls)


def kernel(x, edge_index, batch, W_rel1, b_rel1, W_root1,
           W_rel2, b_rel2, W_root2, W_cls, b_cls):
    npad = _EP - _E
    pad = jnp.concatenate(
        [jnp.zeros((1, npad), jnp.int32),
         jnp.full((1, npad), _N, jnp.int32)], axis=0)
    epad = jnp.concatenate([edge_index, pad], axis=1)
    epad = epad.reshape(2, _EP // _CH, _CH)
    batch2d = batch.reshape(1, _N)

    sc_agg1 = _make_sc_agg(True)
    sc_agg2 = _make_sc_agg(False)

    u0, u1, r1 = _proj_call(x, W_rel1, W_root1, b_rel1.reshape(1, _H))
    s1, c16 = sc_agg1(u0, u1, epad)
    v0, v1, r2 = _mid_call(s1, c16, r1, W_rel2, W_root2, b_rel2.reshape(1, _H))
    s2 = sc_agg2(v0, v1, epad)
    if isinstance(s2, (tuple, list)):
        s2 = s2[0]
    return _pool_call(s2, c16, r2, batch2d, W_cls, b_cls.reshape(1, _H))


# speedup vs baseline: 1.1008x; 1.0525x over previous
import functools

import jax
import jax.numpy as jnp
from jax import lax
from jax.experimental import pallas as pl
from jax.experimental.pallas import tpu as pltpu
from jax.experimental.pallas import tpu_sc as plsc

F32 = jnp.float32

_N = 10000
_E = 320000
_H = 128
_HH = _H // 2
_G = 64

_NC = 2
_NS = 16
_NW = _NC * _NS
_CH = 128
_EP = 327680
_EPT = _EP // _NW
_NCHUNK = _EPT // _CH
_NP = _N + 16
_NH = _N + 16
_RPT = 640
_RCH = 160


def _dot_t(a, w):
    return lax.dot_general(a.astype(jnp.bfloat16), w.astype(jnp.bfloat16),
                           (((1,), (1,)), ((), ())),
                           preferred_element_type=F32)


_NB = 4


def _fill(ref, rows, cols, val):
    groups = cols // 16

    def body(i, c):
        ref[i // groups, pl.ds((i % groups) * 16, 16)] = jnp.full((16,), val, F32)
        return c

    lax.fori_loop(0, rows * groups, body, 0)


def _sc_agg_body(with_counts, *refs):
    if with_counts:
        (u0_hbm, u1_hbm, e_hbm,
         s_out, cnt_out,
         src_blk, dst_blk, zbuf, hist, acc_sp,
         *rest) = refs
    else:
        (u0_hbm, u1_hbm, e_hbm,
         s_out,
         src_blk, dst_blk, zbuf, acc_sp, *rest) = refs
    rows_v = rest[:_NB]
    gsem = rest[_NB:2 * _NB]
    ssem = rest[2 * _NB:3 * _NB]
    csem = rest[3 * _NB:]

    cid = lax.axis_index("c")
    sid = lax.axis_index("s")
    w = cid * _NS + sid
    nck = jnp.clip((_E // _CH) - w * _NCHUNK, _NB, _NCHUNK)
    my_rows = pl.ds(jnp.minimum(sid * _RPT, _N - _RPT), _RPT)

    start_row = jnp.minimum(sid * _RPT, _N - _RPT)

    pltpu.sync_copy(e_hbm.at[0, pl.ds(w * _NCHUNK, _NCHUNK)], src_blk)
    pltpu.sync_copy(e_hbm.at[1, pl.ds(w * _NCHUNK, _NCHUNK)], dst_blk)
    if with_counts:
        def zh(i, c):
            hist[pl.ds(i * 16, 16)] = jnp.zeros((16,), F32)
            return c
        lax.fori_loop(0, _NH // 16, zh, 0)

    for h, u_hbm in ((0, u0_hbm), (1, u1_hbm)):
        count_pass = with_counts and h == 0
        _fill(zbuf, _RCH, _HH, 0.0)
        for k in range(_RPT // _RCH):
            pltpu.sync_copy(zbuf, acc_sp.at[pl.ds(start_row + k * _RCH, _RCH)])
        plsc.subcore_barrier()

        def wait_gather(b):
            pltpu.make_async_copy(u_hbm.at[pl.ds(0, _CH)], rows_v[b],
                                  gsem[b]).wait()

        def start_gather(b, i):
            pltpu.async_copy(u_hbm.at[src_blk.at[i]], rows_v[b], gsem[b])

        def scatter(b, i):
            if count_pass:
                pltpu.async_copy(rows_v[b], acc_sp.at[dst_blk.at[i]], ssem[b],
                                 add=True)
                for g in range(_CH // 16):
                    v = dst_blk[i, pl.ds(g * 16, 16)]
                    cnts, last = plsc.scan_count(v)
                    plsc.addupdate_scatter(hist, [v], cnts.astype(F32),
                                           mask=last)
                pltpu.make_async_copy(rows_v[b], acc_sp.at[pl.ds(0, _CH)],
                                      ssem[b]).wait()
            else:
                pltpu.sync_copy(rows_v[b], acc_sp.at[dst_blk.at[i]], add=True)

        for b in range(_NB):
            start_gather(b, b)

        def window(win, carry):
            for b in range(_NB):
                i = win * _NB + b
                wait_gather(b)
                scatter(b, i)
                start_gather(b, i + _NB)
            return carry

        lax.fori_loop(0, nck // _NB - 1, window, 0)
        for b in range(_NB):
            i = nck - _NB + b
            wait_gather(b)
            scatter(b, i)
        plsc.subcore_barrier()

        for k in range(_RPT // _RCH):
            rows_k = pl.ds(start_row + k * _RCH, _RCH)
            pltpu.sync_copy(acc_sp.at[rows_k], zbuf)
            pltpu.sync_copy(zbuf, s_out.at[cid, rows_k, pl.ds(h * _HH, _HH)])
        if count_pass:
            pltpu.sync_copy(hist.at[pl.ds(0, _N)], cnt_out.at[w])


def _make_sc_agg(with_counts):
    mesh = plsc.VectorSubcoreMesh(core_axis_name="c", subcore_axis_name="s",
                                  num_cores=_NC, num_subcores=_NS)
    out_type = [jax.ShapeDtypeStruct((_NC, _N, _H), F32)]
    scratch = [
        pltpu.VMEM((_NCHUNK, _CH), jnp.int32),
        pltpu.VMEM((_NCHUNK, _CH), jnp.int32),
        pltpu.VMEM((_RCH, _HH), F32),
    ]
    if with_counts:
        out_type.append(jax.ShapeDtypeStruct((_NW, _N), F32))
        scratch.append(pltpu.VMEM((_NH,), F32))
    scratch.append(pltpu.VMEM_SHARED((_NP, _HH), F32))
    scratch += [pltpu.VMEM((_CH, _HH), F32) for _ in range(_NB)]
    scratch += [pltpu.SemaphoreType.DMA for _ in range(3 * _NB)]

    return pl.kernel(
        functools.partial(_sc_agg_body, with_counts),
        out_type=tuple(out_type),
        mesh=mesh,
        scratch_types=scratch,
        compiler_params=pltpu.CompilerParams(use_tc_tiling_on_sc=False,
                                             needs_layout_passes=False),
    )


def _proj_body(x_ref, wrel_ref, wroot_ref, b_ref, u0_ref, u1_ref, r_ref):
    xb = x_ref[...]
    u = _dot_t(xb, wrel_ref[...])
    u0_ref[...] = u[:, :_HH]
    u1_ref[...] = u[:, _HH:]
    r_ref[...] = _dot_t(xb, wroot_ref[...]) + b_ref[...]


def _combine(s_ref, c_ref, r_ref):
    ssum = s_ref[0] + s_ref[1]
    csum = jnp.sum(c_ref[...], axis=0)
    inv = 1.0 / jnp.maximum(csum, 1.0)
    return jnp.maximum(ssum * inv[:, None] + r_ref[...], 0.0)


def _mid_body(s_ref, c16_ref, r_ref, wrel_ref, wroot_ref, b_ref,
              u0_ref, u1_ref, r2_ref):
    h = _combine(s_ref, c16_ref, r_ref)
    u = _dot_t(h, wrel_ref[...])
    u0_ref[...] = u[:, :_HH]
    u1_ref[...] = u[:, _HH:]
    r2_ref[...] = _dot_t(h, wroot_ref[...]) + b_ref[...]


def _pool_body(s_ref, c16_ref, r_ref, batch_ref, wcls_ref, bcls_ref, out_ref):
    h = _combine(s_ref, c16_ref, r_ref)
    gid = lax.broadcasted_iota(jnp.int32, (_G, _N), 0)
    mask = (batch_ref[...] == gid).astype(F32)
    seg = lax.dot_general(mask.astype(jnp.bfloat16), h.astype(jnp.bfloat16),
                          (((1,), (0,)), ((), ())),
                          preferred_element_type=F32)
    cnt = jnp.sum(mask, axis=1)
    pooled = seg / jnp.maximum(cnt, 1.0)[:, None]
    out_ref[...] = _dot_t(pooled, wcls_ref[...]) + bcls_ref[...]


_ROWS_BLK = 2000


def _proj_call(x, wrel, wroot, b):
    n = x.shape[0]
    grid = n // _ROWS_BLK
    full = pl.BlockSpec((_H, _H), lambda i: (0, 0))
    rows = pl.BlockSpec((_ROWS_BLK, _H), lambda i: (i, 0))
    rows_h = pl.BlockSpec((_ROWS_BLK, _HH), lambda i: (i, 0))
    return pl.pallas_call(
        _proj_body,
        grid=(grid,),
        in_specs=[rows, full, full, pl.BlockSpec((1, _H), lambda i: (0, 0))],
        out_specs=[rows_h, rows_h, rows],
        out_shape=[jax.ShapeDtypeStruct((n, _HH), F32)] * 2
        + [jax.ShapeDtypeStruct((n, _H), F32)],
    )(x, wrel, wroot, b)


def _mid_call(s_part, c16, r, wrel, wroot, b):
    return pl.pallas_call(
        _mid_body,
        out_shape=[jax.ShapeDtypeStruct((_N, _HH), F32)] * 2
        + [jax.ShapeDtypeStruct((_N, _H), F32)],
    )(s_part, c16, r, wrel, wroot, b)


def _pool_call(s_part, c16, r, batch2d, wcls, bcls):
    return pl.pallas_call(
        _pool_body,
        out_shape=jax.ShapeDtypeStruct((_G, _H), F32),
    )(s_part, c16, r, batch2d, wcls, bcls)


def kernel(x, edge_index, batch, W_rel1, b_rel1, W_root1,
           W_rel2, b_rel2, W_root2, W_cls, b_cls):
    npad = _EP - _E
    pad = jnp.concatenate(
        [jnp.zeros((1, npad), jnp.int32),
         jnp.full((1, npad), _N, jnp.int32)], axis=0)
    epad = jnp.concatenate([edge_index, pad], axis=1)
    epad = epad.reshape(2, _EP // _CH, _CH)
    batch2d = batch.reshape(1, _N)

    sc_agg1 = _make_sc_agg(True)
    sc_agg2 = _make_sc_agg(False)

    u0, u1, r1 = _proj_call(x, W_rel1, W_root1, b_rel1.reshape(1, _H))
    s1, c16 = sc_agg1(u0, u1, epad)
    v0, v1, r2 = _mid_call(s1, c16, r1, W_rel2, W_root2, b_rel2.reshape(1, _H))
    s2 = sc_agg2(v0, v1, epad)
    if isinstance(s2, (tuple, list)):
        s2 = s2[0]
    return _pool_call(s2, c16, r2, batch2d, W_cls, b_cls.reshape(1, _H))
